# Initial kernel scaffold; baseline (speedup 1.0000x reference)
import functools

import jax
import jax.numpy as jnp
from jax import lax
from jax.experimental import pallas as pl
from jax.experimental.pallas import tpu as pltpu
from jax.experimental.pallas import tpu_sc as plsc


def kernel(seq_len, table):
    del seq_len
    n = (table.shape[0] + 1) // 2
    d = table.shape[1]

    info = plsc.get_sparse_core_info()
    nw = info.num_cores * info.num_subcores
    rows_per_w = n // nw

    mesh = plsc.VectorSubcoreMesh(core_axis_name="c", subcore_axis_name="s")

    @functools.partial(
        pl.kernel,
        mesh=mesh,
        out_type=jax.ShapeDtypeStruct((n, n, d), jnp.float32),
        scratch_types=[
            pltpu.VMEM((2 * n - 1, d), jnp.float32),
            pltpu.SemaphoreType.DMA,
            pltpu.SemaphoreType.DMA,
        ],
    )
    def rel_pos_embed(table_hbm, out_hbm, tab_v, load_sem, store_sem):
        wid = lax.axis_index("s") * info.num_cores + lax.axis_index("c")
        pltpu.async_copy(table_hbm, tab_v, load_sem).wait()
        base = wid * rows_per_w
        copies = []
        for r in range(rows_per_w):
            i = base + r
            start = (n - 1) - i
            copies.append(
                pltpu.async_copy(
                    tab_v.at[pl.ds(start, n)], out_hbm.at[i], store_sem
                )
            )
        for c in copies:
            c.wait()

    return rel_pos_embed(table)


# speedup vs baseline: 5.5177x; 5.5177x over previous
import functools

import jax
import jax.numpy as jnp
from jax import lax
from jax.experimental import pallas as pl
from jax.experimental.pallas import tpu as pltpu
from jax.experimental.pallas import tpu_sc as plsc


def kernel(seq_len, table):
    del seq_len
    n = (table.shape[0] + 1) // 2
    d = table.shape[1]

    info = plsc.get_sparse_core_info()
    nw = info.num_cores * info.num_subcores
    rows_per_w = n // nw
    win_words = (n + rows_per_w - 1) * d
    row_words = n * d

    mesh = plsc.VectorSubcoreMesh(core_axis_name="c", subcore_axis_name="s")

    @functools.partial(
        pl.kernel,
        mesh=mesh,
        out_type=jax.ShapeDtypeStruct((n * n * d,), jnp.float32),
        scratch_types=[
            pltpu.VMEM((win_words,), jnp.float32),
            pltpu.SemaphoreType.DMA,
            pltpu.SemaphoreType.DMA,
        ],
    )
    def rel_pos_embed(table_hbm, out_hbm, tab_v, load_sem, store_sem):
        wid = lax.axis_index("s") * info.num_cores + lax.axis_index("c")
        base = wid * rows_per_w
        src0 = (n - rows_per_w - base) * d
        pltpu.async_copy(
            table_hbm.at[pl.ds(src0, win_words)], tab_v, load_sem
        ).wait()
        copies = []
        for r in range(rows_per_w):
            i = base + r
            local = (rows_per_w - 1 - r) * d
            copies.append(
                pltpu.async_copy(
                    tab_v.at[pl.ds(local, row_words)],
                    out_hbm.at[pl.ds(i * row_words, row_words)],
                    store_sem,
                )
            )
        for c in copies:
            c.wait()

    out = rel_pos_embed(table.reshape(-1))
    return out.reshape(n, n, d)


# speedup vs baseline: 34.1589x; 6.1908x over previous
import functools

import jax
import jax.numpy as jnp
from jax import lax
from jax.experimental import pallas as pl
from jax.experimental.pallas import tpu as pltpu
from jax.experimental.pallas import tpu_sc as plsc


def kernel(seq_len, table):
    del seq_len
    n = (table.shape[0] + 1) // 2
    d = table.shape[1]

    info = plsc.get_sparse_core_info()
    nw = info.num_cores * info.num_subcores
    rows_per_w = n // nw
    sub, lanes = 8, 128
    d_tiles = d // sub
    j_tiles = n // lanes
    band = 8 * rows_per_w
    n_bands = n // band
    win_cols = n + band - 8
    pad_w = 2 * n

    ttp = jnp.pad(table.T, ((0, 0), (0, pad_w + 7 - (2 * n - 1))))
    tts = jnp.stack([ttp[:, k:k + pad_w] for k in range(8)])

    mesh = plsc.VectorSubcoreMesh(core_axis_name="c", subcore_axis_name="s")

    @functools.partial(
        pl.kernel,
        mesh=mesh,
        out_type=jax.ShapeDtypeStruct((n * d_tiles * j_tiles, sub, lanes), jnp.float32),
        scratch_types=[
            pltpu.VMEM((d, win_cols), jnp.float32),
            pltpu.SemaphoreType.DMA,
            pltpu.SemaphoreType.DMA,
        ],
        compiler_params=pltpu.CompilerParams(use_tc_tiling_on_sc=False),
    )
    def rel_pos_embed(tts_hbm, out_hbm, stage, load_sem, store_sem):
        wid = lax.axis_index("s") * info.num_cores + lax.axis_index("c")
        g = wid // 8
        m = wid % 8
        k = 7 - m
        s0 = pl.multiple_of(n - band * (g + 1), 8)
        pltpu.async_copy(
            tts_hbm.at[k, :, pl.ds(s0, win_cols)], stage, load_sem
        ).wait()

        tiles_per_row = d_tiles * j_tiles

        def fire(q):
            i = g * band + 8 * q + m
            t0 = i * tiles_per_row
            c0 = pl.multiple_of(8 * (rows_per_w - 1 - q), 8)
            for td in range(d_tiles):
                for tj in range(j_tiles):
                    pltpu.async_copy(
                        stage.at[pl.ds(td * sub, sub), pl.ds(c0 + tj * lanes, lanes)],
                        out_hbm.at[t0 + td * j_tiles + tj],
                        store_sem,
                    )

        def drain_one_row():
            for _ in range(tiles_per_row):
                pltpu.make_async_copy(
                    out_hbm.at[0], out_hbm.at[0], store_sem
                ).wait()

        def body(q, carry):
            @pl.when(q > 0)
            def _():
                drain_one_row()

            fire(q)
            return carry

        lax.fori_loop(0, rows_per_w, body, None)
        drain_one_row()

    out = rel_pos_embed(tts)
    x = out.reshape(n, d_tiles, j_tiles, sub, lanes)
    return x.transpose(0, 2, 4, 1, 3).reshape(n, n, d)
